# baseline (speedup 1.0000x reference)
import jax
import jax.numpy as jnp
from jax import lax
from jax.experimental import pallas as pl
from jax.experimental.pallas import tpu as pltpu, tpu_sc as plsc

N_TOPICS = 100000
N_CLASSES = 128
BATCH = 16384

NC = 2
NS = 16
NW = NC * NS
LANES = 16

ROWS_PER_W = BATCH // NW
CHUNK = 128
N_CHUNKS = ROWS_PER_W // CHUNK


def _sc_body(logits_hbm, topics_hbm, loga_hbm, b_hbm, out_hbm,
             idx_v, lg_v, lo_v, bb_v,
             sem_idx, sem_b, sem_lg, sem_lo, sem_out):
    wid = lax.axis_index("s") * NC + lax.axis_index("c")
    base0 = wid * ROWS_PER_W

    HALF = CHUNK // 2

    def start_gathers(ci, r0, rn, k):
        s, bs = ci % 2, ci % 3
        idx_s = idx_v.at[ci, pl.ds(r0, rn)]
        cb = pltpu.make_async_copy(b_hbm.at[idx_s],
                                   bb_v.at[bs, pl.ds(r0, rn)], sem_b.at[k])
        cg = pltpu.make_async_copy(loga_hbm.at[idx_s],
                                   lg_v.at[s, pl.ds(r0, rn)], sem_lg.at[k])
        cl = pltpu.make_async_copy(
            logits_hbm.at[pl.ds(base0 + ci * CHUNK + r0, rn)],
            lo_v.at[s, pl.ds(r0, rn)], sem_lo.at[k])
        cb.start(); cg.start(); cl.start()
        return cb, cg, cl

    def start_out(ci, r0, rn, k):
        cp = pltpu.make_async_copy(
            bb_v.at[ci % 3, pl.ds(r0, rn)],
            out_hbm.at[pl.ds(base0 + ci * CHUNK + r0, rn)], sem_out.at[k])
        cp.start()
        return cp

    def compute(ci, r0, rn):
        s, bs = ci % 2, ci % 3
        lo_s, bb_s, lg_s = lo_v.at[s], bb_v.at[bs], lg_v.at[s]

        @plsc.parallel_loop(r0 // LANES, (r0 + rn) // LANES)
        def exp_body(g):
            lg_s[pl.ds(g * LANES, LANES)] = jnp.exp(lg_s[pl.ds(g * LANES, LANES)])

        @plsc.parallel_loop(r0, r0 + rn, unroll=2)
        def row_body(r):
            g = (r >> 4) << 4
            j = r & (LANES - 1)
            svec = lg_s[pl.ds(g, LANES)]
            srow = lax.gather(
                svec, jnp.full((LANES, 1), j, jnp.int32),
                lax.GatherDimensionNumbers(
                    offset_dims=(), collapsed_slice_dims=(0,),
                    start_index_map=(0,)),
                slice_sizes=(1,),
                mode=lax.GatherScatterMode.PROMISE_IN_BOUNDS)
            for k in range(N_CLASSES // LANES):
                sl = pl.ds(k * LANES, LANES)
                bb_s[r, sl] = lo_s[r, sl] * srow + bb_s[r, sl]

    cp_idx = pltpu.make_async_copy(topics_hbm.at[wid], idx_v, sem_idx)
    cp_idx.start()
    cp_idx.wait()

    g0a = start_gathers(0, 0, HALF, 0)
    g0b = start_gathers(0, HALF, HALF, 4)
    g1 = start_gathers(1, 0, CHUNK, 1)
    for cp in g0a:
        cp.wait()
    compute(0, 0, HALF)
    out0a = start_out(0, 0, HALF, 0)
    for cp in g0b:
        cp.wait()
    compute(0, HALF, HALF)
    out0b = start_out(0, HALF, HALF, 4)

    g2 = start_gathers(2, 0, CHUNK, 2)
    for cp in g1:
        cp.wait()
    compute(1, 0, HALF)
    out1a = start_out(1, 0, HALF, 2)
    compute(1, HALF, HALF)
    out1b = start_out(1, HALF, HALF, 3)

    out0a.wait()
    out0b.wait()
    g3 = start_gathers(3, 0, CHUNK, 3)
    for cp in g2:
        cp.wait()
    compute(2, 0, HALF)
    out2a = start_out(2, 0, HALF, 5)
    compute(2, HALF, HALF)
    out2b = start_out(2, HALF, HALF, 6)

    for cp in g3:
        cp.wait()
    compute(3, 0, HALF)
    out3a = start_out(3, 0, HALF, 0)
    compute(3, HALF, HALF)
    out3b = start_out(3, HALF, HALF, 4)

    out1a.wait()
    out1b.wait()
    out2a.wait()
    out2b.wait()
    out3a.wait()
    out3b.wait()


@jax.jit
def _calibrate(logits, topics, loga, b):
    mesh = plsc.VectorSubcoreMesh(
        core_axis_name="c", subcore_axis_name="s",
        num_cores=NC, num_subcores=NS)
    return pl.kernel(
        _sc_body,
        out_type=jax.ShapeDtypeStruct((BATCH, N_CLASSES), jnp.float32),
        mesh=mesh,
        scratch_types=[
            pltpu.VMEM((N_CHUNKS, CHUNK), jnp.int32),
            pltpu.VMEM((2, CHUNK), jnp.float32),
            pltpu.VMEM((2, CHUNK, N_CLASSES), jnp.float32),
            pltpu.VMEM((3, CHUNK, N_CLASSES), jnp.float32),
            pltpu.SemaphoreType.DMA,
            pltpu.SemaphoreType.DMA((5,)),
            pltpu.SemaphoreType.DMA((5,)),
            pltpu.SemaphoreType.DMA((5,)),
            pltpu.SemaphoreType.DMA((7,)),
        ],
    )(logits, topics, loga, b)


def kernel(logits, topics, loga, b):
    topics3 = topics.astype(jnp.int32).reshape(NW, N_CHUNKS, CHUNK)
    return _calibrate(logits, topics3, loga, b)


# speedup vs baseline: 1.0265x; 1.0265x over previous
import jax
import jax.numpy as jnp
from jax import lax
from jax.experimental import pallas as pl
from jax.experimental.pallas import tpu as pltpu, tpu_sc as plsc

N_TOPICS = 100000
N_CLASSES = 128
BATCH = 16384

NC = 2
NS = 16
NW = NC * NS
LANES = 16

ROWS_PER_W = BATCH // NW
CHUNK = 128
N_CHUNKS = ROWS_PER_W // CHUNK


def _sc_body(logits_hbm, topics_hbm, loga_hbm, b_hbm, out_hbm,
             idx_v, lg_v, lo_v, bb_v,
             sem_idx, sem_b, sem_lg, sem_lo, sem_out):
    wid = lax.axis_index("s") * NC + lax.axis_index("c")
    base0 = wid * ROWS_PER_W

    HALF = CHUNK // 2

    def start_gathers(ci, r0, rn, k):
        s, bs = ci % 2, ci % 3
        idx_s = idx_v.at[ci, pl.ds(r0, rn)]
        cb = pltpu.make_async_copy(b_hbm.at[idx_s],
                                   bb_v.at[bs, pl.ds(r0, rn)], sem_b.at[k])
        cg = pltpu.make_async_copy(loga_hbm.at[idx_s],
                                   lg_v.at[s, pl.ds(r0, rn)], sem_lg.at[k])
        cl = pltpu.make_async_copy(
            logits_hbm.at[pl.ds(base0 + ci * CHUNK + r0, rn)],
            lo_v.at[s, pl.ds(r0, rn)], sem_lo.at[k])
        cb.start(); cg.start(); cl.start()
        return cb, cg, cl

    def start_out(ci, r0, rn, k):
        cp = pltpu.make_async_copy(
            bb_v.at[ci % 3, pl.ds(r0, rn)],
            out_hbm.at[pl.ds(base0 + ci * CHUNK + r0, rn)], sem_out.at[k])
        cp.start()
        return cp

    def compute(ci, r0, rn):
        s, bs = ci % 2, ci % 3
        lo_s, bb_s, lg_s = lo_v.at[s], bb_v.at[bs], lg_v.at[s]

        @plsc.parallel_loop(r0 // LANES, (r0 + rn) // LANES)
        def exp_body(g):
            lg_s[pl.ds(g * LANES, LANES)] = jnp.exp(lg_s[pl.ds(g * LANES, LANES)])

        @plsc.parallel_loop(r0, r0 + rn, unroll=2)
        def row_body(r):
            g = (r >> 4) << 4
            j = r & (LANES - 1)
            svec = lg_s[pl.ds(g, LANES)]
            srow = lax.gather(
                svec, jnp.full((LANES, 1), j, jnp.int32),
                lax.GatherDimensionNumbers(
                    offset_dims=(), collapsed_slice_dims=(0,),
                    start_index_map=(0,)),
                slice_sizes=(1,),
                mode=lax.GatherScatterMode.PROMISE_IN_BOUNDS)
            for k in range(N_CLASSES // LANES):
                sl = pl.ds(k * LANES, LANES)
                bb_s[r, sl] = lo_s[r, sl] * srow + bb_s[r, sl]

    cp_idx = pltpu.make_async_copy(topics_hbm.at[wid], idx_v, sem_idx)
    cp_idx.start()
    cp_idx.wait()

    g0a = start_gathers(0, 0, HALF, 0)
    g0b = start_gathers(0, HALF, HALF, 4)
    g1 = start_gathers(1, 0, CHUNK, 1)
    for cp in g0a:
        cp.wait()
    compute(0, 0, HALF)
    out0a = start_out(0, 0, HALF, 0)
    for cp in g0b:
        cp.wait()
    compute(0, HALF, HALF)
    out0b = start_out(0, HALF, HALF, 4)

    g2 = start_gathers(2, 0, CHUNK, 2)
    for cp in g1:
        cp.wait()
    compute(1, 0, CHUNK)
    out1 = start_out(1, 0, CHUNK, 1)

    out0a.wait()
    out0b.wait()
    g3 = start_gathers(3, 0, CHUNK, 3)
    for cp in g2:
        cp.wait()
    compute(2, 0, CHUNK)
    out2 = start_out(2, 0, CHUNK, 2)

    for cp in g3:
        cp.wait()
    compute(3, 0, HALF)
    out3a = start_out(3, 0, HALF, 0)
    compute(3, HALF, HALF)
    out3b = start_out(3, HALF, HALF, 4)

    out1.wait()
    out2.wait()
    out3a.wait()
    out3b.wait()


@jax.jit
def _calibrate(logits, topics, loga, b):
    mesh = plsc.VectorSubcoreMesh(
        core_axis_name="c", subcore_axis_name="s",
        num_cores=NC, num_subcores=NS)
    return pl.kernel(
        _sc_body,
        out_type=jax.ShapeDtypeStruct((BATCH, N_CLASSES), jnp.float32),
        mesh=mesh,
        scratch_types=[
            pltpu.VMEM((N_CHUNKS, CHUNK), jnp.int32),
            pltpu.VMEM((2, CHUNK), jnp.float32),
            pltpu.VMEM((2, CHUNK, N_CLASSES), jnp.float32),
            pltpu.VMEM((3, CHUNK, N_CLASSES), jnp.float32),
            pltpu.SemaphoreType.DMA,
            pltpu.SemaphoreType.DMA((5,)),
            pltpu.SemaphoreType.DMA((5,)),
            pltpu.SemaphoreType.DMA((5,)),
            pltpu.SemaphoreType.DMA((5,)),
        ],
    )(logits, topics, loga, b)


def kernel(logits, topics, loga, b):
    topics3 = topics.astype(jnp.int32).reshape(NW, N_CHUNKS, CHUNK)
    return _calibrate(logits, topics3, loga, b)
